# baseline (speedup 1.0000x reference)
import functools

import jax
import jax.numpy as jnp
from jax import lax
from jax.experimental import pallas as pl
from jax.experimental.pallas import tpu as pltpu
from jax.experimental.pallas import tpu_sc as plsc

_N = 10000
_ND = 5000
_F = 128
_K = 27
_KD = 8
_NW = 32
_NSUB = 16
_EB = 128

_ACC1 = 10112
_RPT1 = 632
_CBLK0 = 80
_CBLK1 = 80

_ACCD = 5120
_RPTD = 320
_DBLK0 = 16
_DBLK1 = 8

_HIGH = lax.Precision.HIGHEST


def _edge_pass(n_nodes, acc_rows, nblk0, nblk1, igrp, rows_per_tile):
  mesh = plsc.VectorSubcoreMesh(core_axis_name="c", subcore_axis_name="s")
  chunks = [(m * _EB, min(_EB, rows_per_tile - m * _EB))
            for m in range((rows_per_tile + _EB - 1) // _EB)]
  assert nblk0 % igrp == 0 and nblk1 % igrp == 0 and igrp % 2 == 0
  ngrp0, ngrp1 = nblk0 // igrp, nblk1 // igrp

  @functools.partial(
      pl.kernel,
      mesh=mesh,
      out_type=jax.ShapeDtypeStruct((2, acc_rows, _F), jnp.float32),
      scratch_types=[
          pltpu.VMEM((igrp, _EB), jnp.int32),
          pltpu.VMEM((igrp, _EB), jnp.int32),
          pltpu.VMEM((_EB, _F), jnp.float32),
          pltpu.VMEM((_EB, _F), jnp.float32),
          pltpu.VMEM_SHARED((acc_rows, _F), jnp.float32),
          pltpu.SemaphoreType.DMA,
          pltpu.SemaphoreType.DMA,
          pltpu.SemaphoreType.DMA,
          pltpu.SemaphoreType.DMA,
      ],
  )
  def pass_(table, src_h, koff_h, dst_h, out, gidx_v, dst_v, rows0, rows1, acc,
            semg0, semg1, sems0, sems1):
    c = lax.axis_index("c")
    s = lax.axis_index("s")

    zv = jnp.zeros((16,), jnp.float32)

    def z_body(i, carry):
      for j in range(_F // 16):
        rows0[i, pl.ds(j * 16, 16)] = zv
      return carry

    lax.fori_loop(jnp.int32(0), jnp.int32(_EB), z_body, 0)
    tile_base = s * rows_per_tile
    for off, size in chunks:
      pltpu.sync_copy(rows0.at[pl.ds(0, size)],
                      acc.at[pl.ds(tile_base + off, size)])
    plsc.subcore_barrier()

    def gi_body(i, carry):
      for j in range(_EB // 16):
        sl = pl.ds(j * 16, 16)
        gidx_v[i, sl] = dst_v[i, sl] * jnp.int32(n_nodes) + gidx_v[i, sl]
      return carry

    def e_body(i2, carry):
      b0 = i2 * 2
      b1 = b0 + 1
      pltpu.make_async_copy(table.at[gidx_v.at[b0]], rows0, semg0).wait()
      pltpu.async_copy(rows0, acc.at[dst_v.at[b0]], sems0, add=True)
      pltpu.make_async_copy(table.at[gidx_v.at[b1]], rows1, semg1).wait()
      pltpu.async_copy(rows1, acc.at[dst_v.at[b1]], sems1, add=True)

      @pl.when(b0 + 2 < igrp)
      def _():
        pltpu.make_async_copy(rows0, acc.at[dst_v.at[b0]], sems0).wait()
        pltpu.async_copy(table.at[gidx_v.at[b0 + 2]], rows0, semg0)

      @pl.when(b1 + 2 < igrp)
      def _():
        pltpu.make_async_copy(rows1, acc.at[dst_v.at[b1]], sems1).wait()
        pltpu.async_copy(table.at[gidx_v.at[b1 + 2]], rows1, semg1)

      return carry

    base_w = jnp.where(c == 0, s * nblk0, 16 * nblk0 + s * nblk1)
    ngrp_w = jnp.where(c == 0, ngrp0, ngrp1).astype(jnp.int32)

    def group_body(g, carry):
      gsl = pl.ds(base_w + g * igrp, igrp)
      pltpu.sync_copy(src_h.at[gsl], gidx_v)
      pltpu.sync_copy(koff_h.at[gsl], dst_v)
      lax.fori_loop(jnp.int32(0), jnp.int32(igrp), gi_body, 0)
      pltpu.sync_copy(dst_h.at[gsl], dst_v)

      pltpu.async_copy(table.at[gidx_v.at[jnp.int32(0)]], rows0, semg0)
      pltpu.async_copy(table.at[gidx_v.at[jnp.int32(1)]], rows1, semg1)
      lax.fori_loop(jnp.int32(0), jnp.int32(igrp // 2), e_body, 0)
      pltpu.make_async_copy(rows0, acc.at[dst_v.at[jnp.int32(igrp - 2)]],
                            sems0).wait()
      pltpu.make_async_copy(rows1, acc.at[dst_v.at[jnp.int32(igrp - 1)]],
                            sems1).wait()
      return carry

    lax.fori_loop(jnp.int32(0), ngrp_w, group_body, 0)
    plsc.subcore_barrier()

    for off, size in chunks:
      sl = pl.ds(tile_base + off, size)
      pltpu.sync_copy(acc.at[sl], out.at[c].at[sl])

  return pass_


@functools.cache
def _build_passes():
  return (_edge_pass(_N, _ACC1, _CBLK0, _CBLK1, 40, _RPT1),
          _edge_pass(_ND, _ACCD, _DBLK0, _DBLK1, 8, _RPTD))


def _prep_edges(src, dst, koff, nbt, n_real, acc_rows):
  cap = nbt * _EB
  pad = cap - src.shape[0]
  pad_src = jnp.arange(pad, dtype=src.dtype) % n_real
  src_p = jnp.concatenate([src, pad_src]).reshape(nbt, _EB)
  koff_p = jnp.pad(koff, (0, pad)).reshape(nbt, _EB)
  pad_dst = n_real + jnp.arange(pad, dtype=dst.dtype) % (acc_rows - n_real)
  dst_p = jnp.concatenate([dst, pad_dst]).reshape(nbt, _EB)
  return src_p, koff_p, dst_p


def _silu(x):
  return x * jax.nn.sigmoid(x)


def _bn(x, gamma, beta):
  mu = jnp.mean(x, axis=0, keepdims=True)
  var = jnp.mean((x - mu) * (x - mu), axis=0, keepdims=True)
  return (x - mu) * lax.rsqrt(var + 1e-5) * gamma + beta


def _mm1(x_F, g1, b1, W1):
  def kfn(x_ref, g_ref, b_ref, w_ref, y_ref, h_scr):
    @pl.when(pl.program_id(0) == 0)
    def _():
      h_scr[...] = _silu(_bn(x_ref[...], g_ref[...], b_ref[...]))

    y_ref[0] = jnp.dot(h_scr[...], w_ref[0])

  return pl.pallas_call(
      kfn,
      grid=(_K,),
      in_specs=[
          pl.BlockSpec((_N, _F), lambda k: (0, 0)),
          pl.BlockSpec((1, _F), lambda k: (0, 0)),
          pl.BlockSpec((1, _F), lambda k: (0, 0)),
          pl.BlockSpec((1, _F, _F), lambda k: (k, 0, 0)),
      ],
      out_specs=pl.BlockSpec((1, _N, _F), lambda k: (k, 0, 0)),
      out_shape=jax.ShapeDtypeStruct((_K, _N, _F), jnp.float32),
      scratch_shapes=[pltpu.VMEM((_N, _F), jnp.float32)],
  )(x_F, g1.reshape(1, _F), b1.reshape(1, _F), W1)


def _mid(p, batch_idx, t, t_W, t_b, g2, b2):
  def kfn(p_ref, bi_ref, t_ref, tw_ref, tb_ref, g_ref, b_ref, h_ref):
    h = p_ref[0, : _N, :] + p_ref[1, : _N, :]
    temb = jnp.dot(_silu(t_ref[...]), tw_ref[...], precision=_HIGH) + tb_ref[...]
    onehot = (bi_ref[...] == lax.broadcasted_iota(jnp.int32, (1, 4), 1))
    h = h + jnp.dot(onehot.astype(jnp.float32), temb, precision=_HIGH)
    h_ref[...] = _silu(_bn(h, g_ref[...], b_ref[...]))

  return pl.pallas_call(
      kfn,
      in_specs=[
          pl.BlockSpec((2, _ACC1, _F), lambda: (0, 0, 0)),
          pl.BlockSpec((_N, 1), lambda: (0, 0)),
          pl.BlockSpec((4, 512), lambda: (0, 0)),
          pl.BlockSpec((512, _F), lambda: (0, 0)),
          pl.BlockSpec((1, _F), lambda: (0, 0)),
          pl.BlockSpec((1, _F), lambda: (0, 0)),
          pl.BlockSpec((1, _F), lambda: (0, 0)),
      ],
      out_specs=pl.BlockSpec((_N, _F), lambda: (0, 0)),
      out_shape=jax.ShapeDtypeStruct((_N, _F), jnp.float32),
      grid=(),
  )(p, batch_idx, t, t_W, t_b.reshape(1, _F), g2.reshape(1, _F),
    b2.reshape(1, _F))


def _mm2(h2s, W2):
  def kfn(h_ref, w_ref, y_ref):
    y_ref[0] = jnp.dot(h_ref[...], w_ref[0])

  return pl.pallas_call(
      kfn,
      grid=(_K,),
      in_specs=[
          pl.BlockSpec((_N, _F), lambda k: (0, 0)),
          pl.BlockSpec((1, _F, _F), lambda k: (k, 0, 0)),
      ],
      out_specs=pl.BlockSpec((1, _N, _F), lambda k: (k, 0, 0)),
      out_shape=jax.ShapeDtypeStruct((_K, _N, _F), jnp.float32),
  )(h2s, W2)


def _mm3(q, x_F, W_down):
  def kfn(q_ref, x_ref, w_ref, y_ref, h_scr):
    @pl.when(pl.program_id(0) == 0)
    def _():
      h_scr[...] = q_ref[0, : _ND, :] + q_ref[1, : _ND, :] + x_ref[...]

    y_ref[0] = jnp.dot(h_scr[...], w_ref[0])

  return pl.pallas_call(
      kfn,
      grid=(_KD,),
      in_specs=[
          pl.BlockSpec((2, _ACC1, _F), lambda k: (0, 0, 0)),
          pl.BlockSpec((_ND, _F), lambda k: (0, 0)),
          pl.BlockSpec((1, _F, _F), lambda k: (k, 0, 0)),
      ],
      out_specs=pl.BlockSpec((1, _ND, _F), lambda k: (k, 0, 0)),
      out_shape=jax.ShapeDtypeStruct((_KD, _ND, _F), jnp.float32),
      scratch_shapes=[pltpu.VMEM((_ND, _F), jnp.float32)],
  )(q, x_F, W_down)


def _final(r, C_down):
  def kfn(r_ref, c_ref, out_ref, cout_ref):
    out_ref[...] = r_ref[0, : _ND, :] + r_ref[1, : _ND, :]
    cc = c_ref[...]
    col = lax.broadcasted_iota(jnp.int32, (_ND, 4), 1)
    cout_ref[...] = jnp.where((col == 0) & (cc > 1), jnp.int32(1), cc)

  return pl.pallas_call(
      kfn,
      in_specs=[
          pl.BlockSpec((2, _ACCD, _F), lambda: (0, 0, 0)),
          pl.BlockSpec((_ND, 4), lambda: (0, 0)),
      ],
      out_specs=[
          pl.BlockSpec((_ND, _F), lambda: (0, 0)),
          pl.BlockSpec((_ND, 4), lambda: (0, 0)),
      ],
      out_shape=[
          jax.ShapeDtypeStruct((_ND, _F), jnp.float32),
          jax.ShapeDtypeStruct((_ND, 4), jnp.int32),
      ],
      grid=(),
  )(r, C_down)


def kernel(x_F, x_C, t, edge_index, edge_koff, down_edge_index, down_koff,
           C_down, bn1_gamma, bn1_beta, W1, t_W, t_b, bn2_gamma, bn2_beta, W2,
           W_down):
  from jax._src.config import enable_x64 as _x64_ctx
  with _x64_ctx(False):
    return _kernel_32(x_F, x_C, t, edge_index, edge_koff, down_edge_index,
                      down_koff, C_down, bn1_gamma, bn1_beta, W1, t_W, t_b,
                      bn2_gamma, bn2_beta, W2, W_down)


def _kernel_32(x_F, x_C, t, edge_index, edge_koff, down_edge_index, down_koff,
               C_down, bn1_gamma, bn1_beta, W1, t_W, t_b, bn2_gamma, bn2_beta,
               W2, W_down):
  src = edge_index[0].astype(jnp.int32)
  dst = edge_index[1].astype(jnp.int32)
  koff = edge_koff.astype(jnp.int32)
  d_src = down_edge_index[0].astype(jnp.int32)
  d_dst = down_edge_index[1].astype(jnp.int32)
  d_koff = down_koff.astype(jnp.int32)

  e1 = _prep_edges(src, dst, koff, 16 * (_CBLK0 + _CBLK1), _N, _ACC1)
  ed = _prep_edges(d_src, d_dst, d_koff, 16 * (_DBLK0 + _DBLK1), _ND, _ACCD)

  conv_pass, down_pass = _build_passes()
  y1 = _mm1(x_F, bn1_gamma, bn1_beta, W1)
  p = conv_pass(y1.reshape(_K * _N, _F), *e1)
  h2s = _mid(p, x_C[:, :1], t, t_W, t_b, bn2_gamma, bn2_beta)
  y2 = _mm2(h2s, W2)
  q = conv_pass(y2.reshape(_K * _N, _F), *e1)
  y3 = _mm3(q, x_F, W_down)
  r = down_pass(y3.reshape(_KD * _ND, _F), *ed)
  out, C_out = _final(r, C_down)
  return out, C_out


# speedup vs baseline: 1.0847x; 1.0847x over previous
import functools

import jax
import jax.numpy as jnp
from jax import lax
from jax.experimental import pallas as pl
from jax.experimental.pallas import tpu as pltpu
from jax.experimental.pallas import tpu_sc as plsc

_N = 10000
_ND = 5000
_F = 128
_K = 27
_KD = 8
_NW = 32
_NSUB = 16
_EB = 128

_ACC1 = 10112
_RPT1 = 632
_CBLK0 = 80
_CBLK1 = 80

_ACCD = 5120
_RPTD = 320
_DBLK0 = 16
_DBLK1 = 8

_HIGH = lax.Precision.HIGHEST


def _edge_pass(n_nodes, acc_rows, nblk0, nblk1, igrp, rows_per_tile):
  mesh = plsc.VectorSubcoreMesh(core_axis_name="c", subcore_axis_name="s")
  chunks = [(m * _EB, min(_EB, rows_per_tile - m * _EB))
            for m in range((rows_per_tile + _EB - 1) // _EB)]
  assert nblk0 % igrp == 0 and nblk1 % igrp == 0 and igrp % 2 == 0
  ngrp0, ngrp1 = nblk0 // igrp, nblk1 // igrp

  @functools.partial(
      pl.kernel,
      mesh=mesh,
      out_type=jax.ShapeDtypeStruct((2, acc_rows, _F), jnp.float32),
      scratch_types=[
          pltpu.VMEM((igrp, _EB), jnp.int32),
          pltpu.VMEM((igrp, _EB), jnp.int32),
          pltpu.VMEM((_EB, _F), jnp.float32),
          pltpu.VMEM((_EB, _F), jnp.float32),
          pltpu.VMEM_SHARED((acc_rows, _F), jnp.float32),
          pltpu.SemaphoreType.DMA,
          pltpu.SemaphoreType.DMA,
      ],
  )
  def pass_(table, src_h, koff_h, dst_h, out, gidx_v, dst_v, rows0, rows1, acc,
            sem0, sem1):
    c = lax.axis_index("c")
    s = lax.axis_index("s")

    zv = jnp.zeros((16,), jnp.float32)

    def z_body(i, carry):
      for j in range(_F // 16):
        rows0[i, pl.ds(j * 16, 16)] = zv
      return carry

    lax.fori_loop(jnp.int32(0), jnp.int32(_EB), z_body, 0)
    tile_base = s * rows_per_tile
    for off, size in chunks:
      pltpu.sync_copy(rows0.at[pl.ds(0, size)],
                      acc.at[pl.ds(tile_base + off, size)])
    plsc.subcore_barrier()

    def gi_body(i, carry):
      for j in range(_EB // 16):
        sl = pl.ds(j * 16, 16)
        gidx_v[i, sl] = dst_v[i, sl] * jnp.int32(n_nodes) + gidx_v[i, sl]
      return carry

    def e_body(i2, carry):
      b0 = i2 * 2
      b1 = b0 + 1
      pltpu.make_async_copy(table.at[gidx_v.at[b0]], rows0, sem0).wait()
      pltpu.async_copy(table.at[gidx_v.at[b1]], rows1, sem1)
      pltpu.sync_copy(rows0, acc.at[dst_v.at[b0]], add=True)
      pltpu.make_async_copy(table.at[gidx_v.at[b1]], rows1, sem1).wait()

      @pl.when(b1 + 1 < igrp)
      def _():
        pltpu.async_copy(table.at[gidx_v.at[b1 + 1]], rows0, sem0)

      pltpu.sync_copy(rows1, acc.at[dst_v.at[b1]], add=True)
      return carry

    base_w = jnp.where(c == 0, s * nblk0, 16 * nblk0 + s * nblk1)
    ngrp_w = jnp.where(c == 0, ngrp0, ngrp1).astype(jnp.int32)

    def group_body(g, carry):
      gsl = pl.ds(base_w + g * igrp, igrp)
      pltpu.sync_copy(src_h.at[gsl], gidx_v)
      pltpu.sync_copy(koff_h.at[gsl], dst_v)
      lax.fori_loop(jnp.int32(0), jnp.int32(igrp), gi_body, 0)
      pltpu.sync_copy(dst_h.at[gsl], dst_v)

      pltpu.async_copy(table.at[gidx_v.at[jnp.int32(0)]], rows0, sem0)
      lax.fori_loop(jnp.int32(0), jnp.int32(igrp // 2), e_body, 0)
      return carry

    lax.fori_loop(jnp.int32(0), ngrp_w, group_body, 0)
    plsc.subcore_barrier()

    for off, size in chunks:
      sl = pl.ds(tile_base + off, size)
      pltpu.sync_copy(acc.at[sl], out.at[c].at[sl])

  return pass_


@functools.cache
def _build_passes():
  return (_edge_pass(_N, _ACC1, _CBLK0, _CBLK1, 40, _RPT1),
          _edge_pass(_ND, _ACCD, _DBLK0, _DBLK1, 8, _RPTD))


def _prep_edges(src, dst, koff, nbt, n_real, acc_rows):
  cap = nbt * _EB
  pad = cap - src.shape[0]
  pad_src = jnp.arange(pad, dtype=src.dtype) % n_real
  src_p = jnp.concatenate([src, pad_src]).reshape(nbt, _EB)
  koff_p = jnp.pad(koff, (0, pad)).reshape(nbt, _EB)
  pad_dst = n_real + jnp.arange(pad, dtype=dst.dtype) % (acc_rows - n_real)
  dst_p = jnp.concatenate([dst, pad_dst]).reshape(nbt, _EB)
  return src_p, koff_p, dst_p


def _silu(x):
  return x * jax.nn.sigmoid(x)


def _bn(x, gamma, beta):
  mu = jnp.mean(x, axis=0, keepdims=True)
  var = jnp.mean((x - mu) * (x - mu), axis=0, keepdims=True)
  return (x - mu) * lax.rsqrt(var + 1e-5) * gamma + beta


def _mm1(x_F, g1, b1, W1):
  def kfn(x_ref, g_ref, b_ref, w_ref, y_ref, h_scr):
    @pl.when(pl.program_id(0) == 0)
    def _():
      h_scr[...] = _silu(_bn(x_ref[...], g_ref[...], b_ref[...]))

    y_ref[0] = jnp.dot(h_scr[...], w_ref[0])

  return pl.pallas_call(
      kfn,
      grid=(_K,),
      in_specs=[
          pl.BlockSpec((_N, _F), lambda k: (0, 0)),
          pl.BlockSpec((1, _F), lambda k: (0, 0)),
          pl.BlockSpec((1, _F), lambda k: (0, 0)),
          pl.BlockSpec((1, _F, _F), lambda k: (k, 0, 0)),
      ],
      out_specs=pl.BlockSpec((1, _N, _F), lambda k: (k, 0, 0)),
      out_shape=jax.ShapeDtypeStruct((_K, _N, _F), jnp.float32),
      scratch_shapes=[pltpu.VMEM((_N, _F), jnp.float32)],
  )(x_F, g1.reshape(1, _F), b1.reshape(1, _F), W1)


def _mm2(p, batch_idx, t, t_W, t_b, g2, b2, W2):
  def kfn(p_ref, bi_ref, t_ref, tw_ref, tb_ref, g_ref, b_ref, w_ref, y_ref,
          h_scr):
    @pl.when(pl.program_id(0) == 0)
    def _():
      temb = jnp.dot(_silu(t_ref[...]), tw_ref[...], precision=_HIGH) + tb_ref[...]
      onehot = (bi_ref[...] == lax.broadcasted_iota(jnp.int32, (1, 4), 1))
      h_scr[...] = (p_ref[0, : _N, :] + p_ref[1, : _N, :]
                    + jnp.dot(onehot.astype(jnp.float32), temb, precision=_HIGH))
      h = h_scr[...]
      mu = jnp.mean(h, axis=0, keepdims=True)
      var = jnp.mean((h - mu) * (h - mu), axis=0, keepdims=True)
      h_scr[...] = _silu((h - mu) * lax.rsqrt(var + 1e-5) * g_ref[...]
                         + b_ref[...])

    y_ref[0] = jnp.dot(h_scr[...], w_ref[0])

  return pl.pallas_call(
      kfn,
      grid=(_K,),
      in_specs=[
          pl.BlockSpec((2, _ACC1, _F), lambda k: (0, 0, 0)),
          pl.BlockSpec((_N, 1), lambda k: (0, 0)),
          pl.BlockSpec((4, 512), lambda k: (0, 0)),
          pl.BlockSpec((512, _F), lambda k: (0, 0)),
          pl.BlockSpec((1, _F), lambda k: (0, 0)),
          pl.BlockSpec((1, _F), lambda k: (0, 0)),
          pl.BlockSpec((1, _F), lambda k: (0, 0)),
          pl.BlockSpec((1, _F, _F), lambda k: (k, 0, 0)),
      ],
      out_specs=pl.BlockSpec((1, _N, _F), lambda k: (k, 0, 0)),
      out_shape=jax.ShapeDtypeStruct((_K, _N, _F), jnp.float32),
      scratch_shapes=[pltpu.VMEM((_N, _F), jnp.float32)],
  )(p, batch_idx, t, t_W, t_b.reshape(1, _F), g2.reshape(1, _F),
    b2.reshape(1, _F), W2)


def _mm3(q, x_F, W_down):
  def kfn(q_ref, x_ref, w_ref, y_ref, h_scr):
    @pl.when(pl.program_id(0) == 0)
    def _():
      h_scr[...] = q_ref[0, : _ND, :] + q_ref[1, : _ND, :] + x_ref[...]

    y_ref[0] = jnp.dot(h_scr[...], w_ref[0])

  return pl.pallas_call(
      kfn,
      grid=(_KD,),
      in_specs=[
          pl.BlockSpec((2, _ACC1, _F), lambda k: (0, 0, 0)),
          pl.BlockSpec((_ND, _F), lambda k: (0, 0)),
          pl.BlockSpec((1, _F, _F), lambda k: (k, 0, 0)),
      ],
      out_specs=pl.BlockSpec((1, _ND, _F), lambda k: (k, 0, 0)),
      out_shape=jax.ShapeDtypeStruct((_KD, _ND, _F), jnp.float32),
      scratch_shapes=[pltpu.VMEM((_ND, _F), jnp.float32)],
  )(q, x_F, W_down)


def _final(r, C_down):
  def kfn(r_ref, c_ref, out_ref, cout_ref):
    out_ref[...] = r_ref[0, : _ND, :] + r_ref[1, : _ND, :]
    cc = c_ref[...]
    col = lax.broadcasted_iota(jnp.int32, (_ND, 4), 1)
    cout_ref[...] = jnp.where((col == 0) & (cc > 1), jnp.int32(1), cc)

  return pl.pallas_call(
      kfn,
      in_specs=[
          pl.BlockSpec((2, _ACCD, _F), lambda: (0, 0, 0)),
          pl.BlockSpec((_ND, 4), lambda: (0, 0)),
      ],
      out_specs=[
          pl.BlockSpec((_ND, _F), lambda: (0, 0)),
          pl.BlockSpec((_ND, 4), lambda: (0, 0)),
      ],
      out_shape=[
          jax.ShapeDtypeStruct((_ND, _F), jnp.float32),
          jax.ShapeDtypeStruct((_ND, 4), jnp.int32),
      ],
      grid=(),
  )(r, C_down)


def kernel(x_F, x_C, t, edge_index, edge_koff, down_edge_index, down_koff,
           C_down, bn1_gamma, bn1_beta, W1, t_W, t_b, bn2_gamma, bn2_beta, W2,
           W_down):
  from jax._src.config import enable_x64 as _x64_ctx
  with _x64_ctx(False):
    return _kernel_32(x_F, x_C, t, edge_index, edge_koff, down_edge_index,
                      down_koff, C_down, bn1_gamma, bn1_beta, W1, t_W, t_b,
                      bn2_gamma, bn2_beta, W2, W_down)


def _kernel_32(x_F, x_C, t, edge_index, edge_koff, down_edge_index, down_koff,
               C_down, bn1_gamma, bn1_beta, W1, t_W, t_b, bn2_gamma, bn2_beta,
               W2, W_down):
  src = edge_index[0].astype(jnp.int32)
  dst = edge_index[1].astype(jnp.int32)
  koff = edge_koff.astype(jnp.int32)
  d_src = down_edge_index[0].astype(jnp.int32)
  d_dst = down_edge_index[1].astype(jnp.int32)
  d_koff = down_koff.astype(jnp.int32)

  e1 = _prep_edges(src, dst, koff, 16 * (_CBLK0 + _CBLK1), _N, _ACC1)
  ed = _prep_edges(d_src, d_dst, d_koff, 16 * (_DBLK0 + _DBLK1), _ND, _ACCD)

  conv_pass, down_pass = _build_passes()
  y1 = _mm1(x_F, bn1_gamma, bn1_beta, W1)
  p = conv_pass(y1.reshape(_K * _N, _F), *e1)
  y2 = _mm2(p, x_C[:, :1], t, t_W, t_b, bn2_gamma, bn2_beta, W2)
  q = conv_pass(y2.reshape(_K * _N, _F), *e1)
  y3 = _mm3(q, x_F, W_down)
  r = down_pass(y3.reshape(_KD * _ND, _F), *ed)
  out, C_out = _final(r, C_down)
  return out, C_out
